# baseline (speedup 1.0000x reference)
import functools

import jax
import jax.numpy as jnp
from jax import lax
from jax.experimental import pallas as pl
from jax.experimental.pallas import tpu as pltpu
from jax.experimental.pallas import tpu_sc as plsc


def _make_sc_kernel(R, H, L, C):
    info = plsc.get_sparse_core_info()
    NC, NS, LANES = info.num_cores, info.num_subcores, info.num_lanes
    NW = NC * NS
    assert R % (NW * C) == 0
    rows_per_w = R // NW
    n_chunks = rows_per_w // C
    assert n_chunks % 2 == 0
    mesh = plsc.VectorSubcoreMesh(core_axis_name="c", subcore_axis_name="s")

    @functools.partial(
        pl.kernel,
        mesh=mesh,
        out_type=jax.ShapeDtypeStruct((R, H), jnp.float32),
        scratch_types=[
            pltpu.VMEM((n_chunks, C * L), jnp.int32),
            pltpu.VMEM((C * L, H // 2), jnp.uint32),
            pltpu.VMEM((C * L, H // 2), jnp.uint32),
            pltpu.VMEM((C, H), jnp.float32),
            pltpu.SemaphoreType.DMA,
            pltpu.SemaphoreType.DMA,
        ],
    )
    def k(idx_hbm, tables_hbm, out_hbm, idx_v, rows0, rows1, acc_v, sem0, sem1):
        wid = lax.axis_index("s") * NC + lax.axis_index("c")
        base = wid * rows_per_w
        rows = (rows0, rows1)
        sems = (sem0, sem1)
        himask = jnp.uint32(0xFFFF0000)

        pltpu.sync_copy(idx_hbm.at[pl.ds(wid * n_chunks, n_chunks)], idx_v)

        def fire(ci, buf):
            pltpu.async_copy(tables_hbm.at[idx_v.at[ci]], rows[buf], sems[buf])

        def drain(ci, buf):
            pltpu.make_async_copy(
                tables_hbm.at[idx_v.at[ci]], rows[buf], sems[buf]
            ).wait()

        def consume(ci, buf):
            rv = rows[buf]

            def g_body(gi, carry):
                offu = pl.multiple_of(gi * LANES, LANES)
                off = pl.multiple_of(gi * 2 * LANES, 2 * LANES)
                for c in range(C):
                    s = rv[c * L, pl.ds(offu, LANES)]
                    lo = lax.bitcast_convert_type(s << 16, jnp.float32)
                    hi = lax.bitcast_convert_type(s & himask, jnp.float32)
                    for l in range(1, L):
                        s = rv[c * L + l, pl.ds(offu, LANES)]
                        lo = lo + lax.bitcast_convert_type(s << 16, jnp.float32)
                        hi = hi + lax.bitcast_convert_type(s & himask, jnp.float32)
                    acc_v[c, pl.ds(off, LANES)] = lo
                    acc_v[c, pl.ds(off + LANES, LANES)] = hi
                return carry

            lax.fori_loop(0, H // (2 * LANES), g_body, 0)
            pltpu.sync_copy(acc_v, out_hbm.at[pl.ds(base + ci * C, C)])

        fire(0, 0)

        def pair(cj, carry):
            ci0 = cj * 2
            fire(ci0 + 1, 1)
            drain(ci0, 0)
            consume(ci0, 0)

            @pl.when(ci0 + 2 < n_chunks)
            def _():
                fire(ci0 + 2, 0)

            drain(ci0 + 1, 1)
            consume(ci0 + 1, 1)
            return carry

        lax.fori_loop(0, n_chunks // 2, pair, 0)

    return k


def kernel(input_ids, tables):
    B, L, S = input_ids.shape
    _, V, H = tables.shape
    R = B * S
    C = 8
    ids32 = input_ids.astype(jnp.int32)
    flat_idx = ids32 + (jnp.arange(L, dtype=jnp.int32) * V)[None, :, None]
    flat_idx = flat_idx.transpose(0, 2, 1).reshape(R // C, C * L)
    tab = tables.reshape(L * V, H).astype(jnp.bfloat16)
    tab = tab.reshape(L * V, H // 32, 2, 16).swapaxes(2, 3).reshape(L * V, H // 2, 2)
    tab = lax.bitcast_convert_type(tab, jnp.uint32)
    out = _make_sc_kernel(R, H, L, C)(flat_idx, tab)
    return out.reshape(B, S, H)


# speedup vs baseline: 1.1185x; 1.1185x over previous
import functools

import jax
import jax.numpy as jnp
from jax import lax
from jax.experimental import pallas as pl
from jax.experimental.pallas import tpu as pltpu
from jax.experimental.pallas import tpu_sc as plsc


def _make_sc_kernel(R, H, L, C):
    info = plsc.get_sparse_core_info()
    NC, NS, LANES = info.num_cores, info.num_subcores, info.num_lanes
    NW = NC * NS
    assert R % (NW * C) == 0
    rows_per_w = R // NW
    n_chunks = rows_per_w // C
    assert n_chunks % 2 == 0
    mesh = plsc.VectorSubcoreMesh(core_axis_name="c", subcore_axis_name="s")

    @functools.partial(
        pl.kernel,
        mesh=mesh,
        out_type=jax.ShapeDtypeStruct((R, H), jnp.float32),
        scratch_types=[
            pltpu.VMEM((n_chunks, C * L), jnp.int32),
            pltpu.VMEM((C * L, H // 2), jnp.uint32),
            pltpu.VMEM((C * L, H // 2), jnp.uint32),
            pltpu.VMEM((C, H), jnp.float32),
            pltpu.SemaphoreType.DMA,
            pltpu.SemaphoreType.DMA,
        ],
    )
    def k(idx_hbm, tables_hbm, out_hbm, idx_v, rows0, rows1, acc_v, sem0, sem1):
        wid = lax.axis_index("s") * NC + lax.axis_index("c")
        base = wid * rows_per_w
        rows = (rows0, rows1)
        sems = (sem0, sem1)
        himask = jnp.uint32(0xFFFF0000)

        pltpu.sync_copy(idx_hbm.at[pl.ds(wid * n_chunks, n_chunks)], idx_v)

        def fire(ci, buf):
            pltpu.async_copy(tables_hbm.at[idx_v.at[ci]], rows[buf], sems[buf])

        def drain(ci, buf):
            pltpu.make_async_copy(
                tables_hbm.at[idx_v.at[ci]], rows[buf], sems[buf]
            ).wait()

        def consume(ci, buf):
            rv = rows[buf]

            def g_body(gi, carry):
                offu = pl.multiple_of(gi * LANES, LANES)
                off = pl.multiple_of(gi * 2 * LANES, 2 * LANES)
                for c in range(C):
                    s = rv[c * L, pl.ds(offu, LANES)]
                    lo = lax.bitcast_convert_type(s << 16, jnp.float32)
                    hi = lax.bitcast_convert_type(s & himask, jnp.float32)
                    for l in range(1, L):
                        s = rv[c * L + l, pl.ds(offu, LANES)]
                        lo = lo + lax.bitcast_convert_type(s << 16, jnp.float32)
                        hi = hi + lax.bitcast_convert_type(s & himask, jnp.float32)
                    acc_v[c, pl.ds(off, LANES)] = lo
                    acc_v[c, pl.ds(off + LANES, LANES)] = hi
                return carry

            lax.fori_loop(0, H // (2 * LANES), g_body, 0)
            pltpu.sync_copy(acc_v, out_hbm.at[pl.ds(base + ci * C, C)])

        fire(0, 0)

        def pair(cj, carry):
            ci0 = cj * 2
            fire(ci0 + 1, 1)
            drain(ci0, 0)
            consume(ci0, 0)

            @pl.when(ci0 + 2 < n_chunks)
            def _():
                fire(ci0 + 2, 0)

            drain(ci0 + 1, 1)
            consume(ci0 + 1, 1)
            return carry

        lax.fori_loop(0, n_chunks // 2, pair, 0)

    return k


def kernel(input_ids, tables):
    B, L, S = input_ids.shape
    _, V, H = tables.shape
    R = B * S
    C = 8
    ids32 = input_ids.astype(jnp.int32)
    flat_idx = ids32 + (jnp.arange(L, dtype=jnp.int32) * V)[None, :, None]
    flat_idx = flat_idx.transpose(0, 2, 1).reshape(R // C, C * L)
    tb = tables.reshape(L * V, H).astype(jnp.bfloat16)
    tb = tb.reshape(L * V, H // 32, 2, 16)
    au = lax.bitcast_convert_type(tb[:, :, 0, :], jnp.uint16).astype(jnp.uint32)
    bu = lax.bitcast_convert_type(tb[:, :, 1, :], jnp.uint16).astype(jnp.uint32)
    tab = (au | (bu << 16)).reshape(L * V, H // 2)
    out = _make_sc_kernel(R, H, L, C)(flat_idx, tab)
    return out.reshape(B, S, H)


# speedup vs baseline: 1.8544x; 1.6579x over previous
import functools

import jax
import jax.numpy as jnp
from jax import lax
from jax.experimental import pallas as pl
from jax.experimental.pallas import tpu as pltpu
from jax.experimental.pallas import tpu_sc as plsc


def _make_sc_kernel(R, H, L, C):
    info = plsc.get_sparse_core_info()
    NC, NS, LANES = info.num_cores, info.num_subcores, info.num_lanes
    NW = NC * NS
    assert R % (NW * C) == 0
    rows_per_w = R // NW
    n_chunks = rows_per_w // C
    assert n_chunks % 2 == 0
    mesh = plsc.VectorSubcoreMesh(core_axis_name="c", subcore_axis_name="s")

    @functools.partial(
        pl.kernel,
        mesh=mesh,
        out_type=jax.ShapeDtypeStruct((R, H), jnp.float32),
        scratch_types=[
            pltpu.VMEM((n_chunks, C * L), jnp.int32),
            pltpu.VMEM((C * L, H // 2), jnp.uint32),
            pltpu.VMEM((C * L, H // 2), jnp.uint32),
            pltpu.VMEM((C, H), jnp.float32),
            pltpu.SemaphoreType.DMA,
            pltpu.SemaphoreType.DMA,
        ],
    )
    def k(idx_hbm, tables_hbm, out_hbm, idx_v, rows0, rows1, acc_v, sem0, sem1):
        wid = lax.axis_index("s") * NC + lax.axis_index("c")
        base = wid * rows_per_w
        rows = (rows0, rows1)
        sems = (sem0, sem1)
        himask = jnp.uint32(0xFFFF0000)

        pltpu.sync_copy(idx_hbm.at[pl.ds(wid * n_chunks, n_chunks)], idx_v)

        def fire(ci, buf):
            pltpu.async_copy(tables_hbm.at[idx_v.at[ci]], rows[buf], sems[buf])

        def drain(ci, buf):
            pltpu.make_async_copy(
                tables_hbm.at[idx_v.at[ci]], rows[buf], sems[buf]
            ).wait()

        def consume(ci, buf):
            rv = rows[buf]

            def g_body(gi, carry):
                offu = pl.multiple_of(gi * LANES, LANES)
                for c in range(C):
                    s = rv[c * L, pl.ds(offu, LANES)]
                    lo = lax.bitcast_convert_type(s << 16, jnp.float32)
                    hi = lax.bitcast_convert_type(s & himask, jnp.float32)
                    for l in range(1, L):
                        s = rv[c * L + l, pl.ds(offu, LANES)]
                        lo = lo + lax.bitcast_convert_type(s << 16, jnp.float32)
                        hi = hi + lax.bitcast_convert_type(s & himask, jnp.float32)
                    acc_v[c, pl.ds(offu, LANES)] = lo
                    acc_v[c, pl.ds(offu + H // 2, LANES)] = hi
                return carry

            lax.fori_loop(0, H // (2 * LANES), g_body, 0)
            pltpu.sync_copy(acc_v, out_hbm.at[pl.ds(base + ci * C, C)])

        fire(0, 0)

        def pair(cj, carry):
            ci0 = cj * 2
            fire(ci0 + 1, 1)
            drain(ci0, 0)
            consume(ci0, 0)

            @pl.when(ci0 + 2 < n_chunks)
            def _():
                fire(ci0 + 2, 0)

            drain(ci0 + 1, 1)
            consume(ci0 + 1, 1)
            return carry

        lax.fori_loop(0, n_chunks // 2, pair, 0)

    return k


def _pack_tables(tables, L, V, H):
    u = lax.bitcast_convert_type(tables.reshape(L * V, H), jnp.uint32)

    def rtne(x):
        return (x + jnp.uint32(0x7FFF) + ((x >> 16) & jnp.uint32(1))) >> 16

    return rtne(u[:, : H // 2]) | (rtne(u[:, H // 2 :]) << 16)


def kernel(input_ids, tables):
    B, L, S = input_ids.shape
    _, V, H = tables.shape
    R = B * S
    C = 8
    ids32 = input_ids.astype(jnp.int32)
    flat_idx = ids32 + (jnp.arange(L, dtype=jnp.int32) * V)[None, :, None]
    flat_idx = flat_idx.transpose(0, 2, 1).reshape(R // C, C * L)
    tab = _pack_tables(tables, L, V, H)
    out = _make_sc_kernel(R, H, L, C)(flat_idx, tab)
    return out.reshape(B, S, H)


# speedup vs baseline: 2.0036x; 1.0804x over previous
import functools

import jax
import jax.numpy as jnp
from jax import lax
from jax.experimental import pallas as pl
from jax.experimental.pallas import tpu as pltpu
from jax.experimental.pallas import tpu_sc as plsc


def _make_sc_kernel(R, H, L, C):
    info = plsc.get_sparse_core_info()
    NC, NS, LANES = info.num_cores, info.num_subcores, info.num_lanes
    NW = NC * NS
    assert R % (NW * C) == 0
    rows_per_w = R // NW
    n_chunks = rows_per_w // C
    assert n_chunks % 2 == 0
    mesh = plsc.VectorSubcoreMesh(core_axis_name="c", subcore_axis_name="s")

    @functools.partial(
        pl.kernel,
        mesh=mesh,
        out_type=jax.ShapeDtypeStruct((R, H), jnp.float32),
        scratch_types=[
            pltpu.VMEM((n_chunks, C * L), jnp.int32),
            pltpu.VMEM((C * L, H // 2), jnp.uint32),
            pltpu.VMEM((C * L, H // 2), jnp.uint32),
            pltpu.VMEM((C, H), jnp.float32),
            pltpu.SemaphoreType.DMA,
            pltpu.SemaphoreType.DMA,
        ],
    )
    def k(idx_hbm, tables_hbm, out_hbm, idx_v, rows0, rows1, acc_v, sem0, sem1):
        wid = lax.axis_index("s") * NC + lax.axis_index("c")
        base = wid * rows_per_w
        rows = (rows0, rows1)
        sems = (sem0, sem1)
        himask = jnp.uint32(0xFFFF0000)

        pltpu.sync_copy(idx_hbm.at[pl.ds(wid * n_chunks, n_chunks)], idx_v)

        def fire(ci, buf):
            pltpu.async_copy(tables_hbm.at[idx_v.at[ci]], rows[buf], sems[buf])

        def drain(ci, buf):
            pltpu.make_async_copy(
                tables_hbm.at[idx_v.at[ci]], rows[buf], sems[buf]
            ).wait()

        def consume(ci, buf):
            rv = rows[buf]

            def g_body(gi, carry):
                offu = pl.multiple_of(gi * LANES, LANES)
                for c in range(C):
                    s = rv[c * L, pl.ds(offu, LANES)]
                    lo = lax.bitcast_convert_type(s << 16, jnp.float32)
                    hi = lax.bitcast_convert_type(s & himask, jnp.float32)
                    for l in range(1, L):
                        s = rv[c * L + l, pl.ds(offu, LANES)]
                        lo = lo + lax.bitcast_convert_type(s << 16, jnp.float32)
                        hi = hi + lax.bitcast_convert_type(s & himask, jnp.float32)
                    acc_v[c, pl.ds(offu, LANES)] = lo
                    acc_v[c, pl.ds(offu + H // 2, LANES)] = hi
                return carry

            lax.fori_loop(0, H // (2 * LANES), g_body, 0)
            pltpu.sync_copy(acc_v, out_hbm.at[pl.ds(base + ci * C, C)])

        fire(0, 0)

        def pair(cj, carry):
            ci0 = cj * 2
            fire(ci0 + 1, 1)
            drain(ci0, 0)
            consume(ci0, 0)

            @pl.when(ci0 + 2 < n_chunks)
            def _():
                fire(ci0 + 2, 0)

            drain(ci0 + 1, 1)
            consume(ci0 + 1, 1)
            return carry

        lax.fori_loop(0, n_chunks // 2, pair, 0)

    return k


def _pack_tables(tables, L, V, H):
    rows = L * V
    blk = 256

    def body(t_ref, o_ref):
        u = lax.bitcast_convert_type(t_ref[...], jnp.uint32)

        def rtne(x):
            return (x + jnp.uint32(0x7FFF) + ((x >> 16) & jnp.uint32(1))) >> 16

        o_ref[...] = rtne(u[:, : H // 2]) | (rtne(u[:, H // 2 :]) << 16)

    return pl.pallas_call(
        body,
        out_shape=jax.ShapeDtypeStruct((rows, H // 2), jnp.uint32),
        grid=(rows // blk,),
        in_specs=[pl.BlockSpec((blk, H), lambda i: (i, 0))],
        out_specs=pl.BlockSpec((blk, H // 2), lambda i: (i, 0)),
    )(tables.reshape(rows, H))


def kernel(input_ids, tables):
    B, L, S = input_ids.shape
    _, V, H = tables.shape
    R = B * S
    C = 8
    ids32 = input_ids.astype(jnp.int32)
    flat_idx = ids32 + (jnp.arange(L, dtype=jnp.int32) * V)[None, :, None]
    flat_idx = flat_idx.transpose(0, 2, 1).reshape(R // C, C * L)
    tab = _pack_tables(tables, L, V, H)
    out = _make_sc_kernel(R, H, L, C)(flat_idx, tab)
    return out.reshape(B, S, H)


# speedup vs baseline: 2.1286x; 1.0624x over previous
import functools

import jax
import jax.numpy as jnp
from jax import lax
from jax.experimental import pallas as pl
from jax.experimental.pallas import tpu as pltpu
from jax.experimental.pallas import tpu_sc as plsc


def _make_sc_kernel(R, H, L, C):
    info = plsc.get_sparse_core_info()
    NC, NS, LANES = info.num_cores, info.num_subcores, info.num_lanes
    NW = NC * NS
    assert R % (NW * C) == 0
    rows_per_w = R // NW
    n_chunks = rows_per_w // C
    assert n_chunks % 2 == 0
    mesh = plsc.VectorSubcoreMesh(core_axis_name="c", subcore_axis_name="s")

    @functools.partial(
        pl.kernel,
        mesh=mesh,
        out_type=jax.ShapeDtypeStruct((R, H), jnp.float32),
        scratch_types=[
            pltpu.VMEM((n_chunks, C * L), jnp.int32),
            pltpu.VMEM((C * L, H // 2), jnp.uint32),
            pltpu.VMEM((C * L, H // 2), jnp.uint32),
            pltpu.VMEM((C, H), jnp.float32),
            pltpu.SemaphoreType.DMA,
            pltpu.SemaphoreType.DMA,
        ],
    )
    def k(idx_hbm, tables_hbm, out_hbm, idx_v, rows0, rows1, acc_v, sem0, sem1):
        wid = lax.axis_index("s") * NC + lax.axis_index("c")
        base = wid * rows_per_w
        rows = (rows0, rows1)
        sems = (sem0, sem1)
        himask = jnp.uint32(0xFFFF0000)

        pltpu.sync_copy(idx_hbm.at[pl.ds(wid * n_chunks, n_chunks)], idx_v)

        def fire(ci, buf):
            pltpu.async_copy(tables_hbm.at[idx_v.at[ci]], rows[buf], sems[buf])

        def drain(ci, buf):
            pltpu.make_async_copy(
                tables_hbm.at[idx_v.at[ci]], rows[buf], sems[buf]
            ).wait()

        def consume(ci, buf):
            rv = rows[buf]

            def g_body(gi, carry):
                offu = pl.multiple_of(gi * LANES, LANES)
                for c in range(C):
                    s = rv[c * L, pl.ds(offu, LANES)]
                    lo = lax.bitcast_convert_type(s << 16, jnp.float32)
                    hi = lax.bitcast_convert_type(s & himask, jnp.float32)
                    for l in range(1, L):
                        s = rv[c * L + l, pl.ds(offu, LANES)]
                        lo = lo + lax.bitcast_convert_type(s << 16, jnp.float32)
                        hi = hi + lax.bitcast_convert_type(s & himask, jnp.float32)
                    acc_v[c, pl.ds(offu, LANES)] = lo
                    acc_v[c, pl.ds(offu + H // 2, LANES)] = hi
                return carry

            lax.fori_loop(0, H // (2 * LANES), g_body, 0)
            pltpu.sync_copy(acc_v, out_hbm.at[pl.ds(base + ci * C, C)])

        fire(0, 0)

        def pair(cj, carry):
            ci0 = cj * 2
            fire(ci0 + 1, 1)
            drain(ci0, 0)
            consume(ci0, 0)

            @pl.when(ci0 + 2 < n_chunks)
            def _():
                fire(ci0 + 2, 0)

            drain(ci0 + 1, 1)
            consume(ci0 + 1, 1)
            return carry

        lax.fori_loop(0, n_chunks // 2, pair, 0)

    return k


def _pack_tables(tables, L, V, H):
    rows = L * V
    blk = 512

    def body(t_ref, o_ref):
        u = lax.bitcast_convert_type(t_ref[...], jnp.uint32)

        def rtne(x):
            return (x + jnp.uint32(0x7FFF) + ((x >> 16) & jnp.uint32(1))) >> 16

        o_ref[...] = rtne(u[:, : H // 2]) | (rtne(u[:, H // 2 :]) << 16)

    return pl.pallas_call(
        body,
        out_shape=jax.ShapeDtypeStruct((rows, H // 2), jnp.uint32),
        grid=(rows // blk,),
        in_specs=[pl.BlockSpec((blk, H), lambda i: (i, 0))],
        out_specs=pl.BlockSpec((blk, H // 2), lambda i: (i, 0)),
    )(tables.reshape(rows, H))


def kernel(input_ids, tables):
    B, L, S = input_ids.shape
    _, V, H = tables.shape
    R = B * S
    C = 8
    ids32 = input_ids.astype(jnp.int32)
    flat_idx = ids32 + (jnp.arange(L, dtype=jnp.int32) * V)[None, :, None]
    flat_idx = flat_idx.transpose(0, 2, 1).reshape(R // C, C * L)
    tab = _pack_tables(tables, L, V, H)
    out = _make_sc_kernel(R, H, L, C)(flat_idx, tab)
    return out.reshape(B, S, H)


# speedup vs baseline: 2.1845x; 1.0262x over previous
import functools

import jax
import jax.numpy as jnp
from jax import lax
from jax.experimental import pallas as pl
from jax.experimental.pallas import tpu as pltpu
from jax.experimental.pallas import tpu_sc as plsc


def _make_sc_kernel(R, H, L, C):
    info = plsc.get_sparse_core_info()
    NC, NS, LANES = info.num_cores, info.num_subcores, info.num_lanes
    NW = NC * NS
    assert R % (NW * C) == 0
    rows_per_w = R // NW
    n_chunks = rows_per_w // C
    assert n_chunks % 2 == 0
    mesh = plsc.VectorSubcoreMesh(core_axis_name="c", subcore_axis_name="s")

    @functools.partial(
        pl.kernel,
        mesh=mesh,
        out_type=jax.ShapeDtypeStruct((R, H), jnp.float32),
        scratch_types=[
            pltpu.VMEM((n_chunks, C * L), jnp.int32),
            pltpu.VMEM((C * L, H // 2), jnp.uint32),
            pltpu.VMEM((C * L, H // 2), jnp.uint32),
            pltpu.VMEM((C, H), jnp.float32),
            pltpu.SemaphoreType.DMA,
            pltpu.SemaphoreType.DMA,
        ],
    )
    def k(idx_hbm, tables_hbm, out_hbm, idx_v, rows0, rows1, acc_v, sem0, sem1):
        wid = lax.axis_index("s") * NC + lax.axis_index("c")
        base = wid * rows_per_w
        rows = (rows0, rows1)
        sems = (sem0, sem1)
        himask = jnp.uint32(0xFFFF0000)

        pltpu.sync_copy(idx_hbm.at[pl.ds(wid * n_chunks, n_chunks)], idx_v)

        def fire(ci, buf):
            pltpu.async_copy(tables_hbm.at[idx_v.at[ci]], rows[buf], sems[buf])

        def drain(ci, buf):
            pltpu.make_async_copy(
                tables_hbm.at[idx_v.at[ci]], rows[buf], sems[buf]
            ).wait()

        def consume(ci, buf):
            rv = rows[buf]

            def g_body(gi, carry):
                offu = pl.multiple_of(gi * LANES, LANES)
                for c in range(C):
                    s = rv[c * L, pl.ds(offu, LANES)]
                    lo = lax.bitcast_convert_type(s << 16, jnp.float32)
                    hi = lax.bitcast_convert_type(s & himask, jnp.float32)
                    for l in range(1, L):
                        s = rv[c * L + l, pl.ds(offu, LANES)]
                        lo = lo + lax.bitcast_convert_type(s << 16, jnp.float32)
                        hi = hi + lax.bitcast_convert_type(s & himask, jnp.float32)
                    acc_v[c, pl.ds(offu, LANES)] = lo
                    acc_v[c, pl.ds(offu + H // 2, LANES)] = hi
                return carry

            lax.fori_loop(0, H // (2 * LANES), g_body, 0)
            pltpu.sync_copy(acc_v, out_hbm.at[pl.ds(base + ci * C, C)])

        fire(0, 0)

        def pair(cj, carry):
            ci0 = cj * 2
            fire(ci0 + 1, 1)
            drain(ci0, 0)
            consume(ci0, 0)

            @pl.when(ci0 + 2 < n_chunks)
            def _():
                fire(ci0 + 2, 0)

            drain(ci0 + 1, 1)
            consume(ci0 + 1, 1)
            return carry

        lax.fori_loop(0, n_chunks // 2, pair, 0)

    return k


def _pack_tables(tables, L, V, H):
    rows = L * V
    blk = 1024

    def body(t_ref, o_ref):
        u = lax.bitcast_convert_type(t_ref[...], jnp.uint32)

        def rtne(x):
            return (x + jnp.uint32(0x7FFF) + ((x >> 16) & jnp.uint32(1))) >> 16

        o_ref[...] = rtne(u[:, : H // 2]) | (rtne(u[:, H // 2 :]) << 16)

    return pl.pallas_call(
        body,
        out_shape=jax.ShapeDtypeStruct((rows, H // 2), jnp.uint32),
        grid=(rows // blk,),
        in_specs=[pl.BlockSpec((blk, H), lambda i: (i, 0))],
        out_specs=pl.BlockSpec((blk, H // 2), lambda i: (i, 0)),
    )(tables.reshape(rows, H))


def kernel(input_ids, tables):
    B, L, S = input_ids.shape
    _, V, H = tables.shape
    R = B * S
    C = 8
    ids32 = input_ids.astype(jnp.int32)
    flat_idx = ids32 + (jnp.arange(L, dtype=jnp.int32) * V)[None, :, None]
    flat_idx = flat_idx.transpose(0, 2, 1).reshape(R // C, C * L)
    tab = _pack_tables(tables, L, V, H)
    out = _make_sc_kernel(R, H, L, C)(flat_idx, tab)
    return out.reshape(B, S, H)


# speedup vs baseline: 2.2077x; 1.0106x over previous
import functools

import jax
import jax.numpy as jnp
from jax import lax
from jax.experimental import pallas as pl
from jax.experimental.pallas import tpu as pltpu
from jax.experimental.pallas import tpu_sc as plsc


def _make_sc_kernel(R, H, L, C):
    info = plsc.get_sparse_core_info()
    NC, NS, LANES = info.num_cores, info.num_subcores, info.num_lanes
    NW = NC * NS
    assert R % (NW * C) == 0
    rows_per_w = R // NW
    n_chunks = rows_per_w // C
    assert n_chunks % 2 == 0
    mesh = plsc.VectorSubcoreMesh(core_axis_name="c", subcore_axis_name="s")

    @functools.partial(
        pl.kernel,
        mesh=mesh,
        out_type=jax.ShapeDtypeStruct((R, H), jnp.float32),
        scratch_types=[
            pltpu.VMEM((n_chunks, C * L), jnp.int32),
            pltpu.VMEM((C * L, H // 2), jnp.uint32),
            pltpu.VMEM((C * L, H // 2), jnp.uint32),
            pltpu.VMEM((C, H), jnp.float32),
            pltpu.SemaphoreType.DMA,
            pltpu.SemaphoreType.DMA,
        ],
    )
    def k(idx_hbm, tables_hbm, out_hbm, idx_v, rows0, rows1, acc_v, sem0, sem1):
        wid = lax.axis_index("s") * NC + lax.axis_index("c")
        base = wid * rows_per_w
        rows = (rows0, rows1)
        sems = (sem0, sem1)
        himask = jnp.uint32(0xFFFF0000)

        pltpu.sync_copy(idx_hbm.at[pl.ds(wid * n_chunks, n_chunks)], idx_v)

        def fire(ci, buf):
            pltpu.async_copy(tables_hbm.at[idx_v.at[ci]], rows[buf], sems[buf])

        def drain(ci, buf):
            pltpu.make_async_copy(
                tables_hbm.at[idx_v.at[ci]], rows[buf], sems[buf]
            ).wait()

        def consume(ci, buf):
            rv = rows[buf]

            def g_body(gi, carry):
                offu = pl.multiple_of(gi * LANES, LANES)
                for c in range(C):
                    s = rv[c * L, pl.ds(offu, LANES)]
                    lo = lax.bitcast_convert_type(s << 16, jnp.float32)
                    hi = lax.bitcast_convert_type(s & himask, jnp.float32)
                    for l in range(1, L):
                        s = rv[c * L + l, pl.ds(offu, LANES)]
                        lo = lo + lax.bitcast_convert_type(s << 16, jnp.float32)
                        hi = hi + lax.bitcast_convert_type(s & himask, jnp.float32)
                    acc_v[c, pl.ds(offu, LANES)] = lo
                    acc_v[c, pl.ds(offu + H // 2, LANES)] = hi
                return carry

            lax.fori_loop(0, H // (2 * LANES), g_body, 0)
            pltpu.sync_copy(acc_v, out_hbm.at[pl.ds(base + ci * C, C)])

        fire(0, 0)

        def pair(cj, carry):
            ci0 = cj * 2
            fire(ci0 + 1, 1)
            drain(ci0, 0)
            consume(ci0, 0)

            @pl.when(ci0 + 2 < n_chunks)
            def _():
                fire(ci0 + 2, 0)

            drain(ci0 + 1, 1)
            consume(ci0 + 1, 1)
            return carry

        lax.fori_loop(0, n_chunks // 2, pair, 0)

    return k


def _pack_tables(tables, L, V, H):
    rows = L * V
    blk = 2048

    def body(t_ref, o_ref):
        u = lax.bitcast_convert_type(t_ref[...], jnp.uint32)

        def rtne(x):
            return (x + jnp.uint32(0x7FFF) + ((x >> 16) & jnp.uint32(1))) >> 16

        o_ref[...] = rtne(u[:, : H // 2]) | (rtne(u[:, H // 2 :]) << 16)

    return pl.pallas_call(
        body,
        out_shape=jax.ShapeDtypeStruct((rows, H // 2), jnp.uint32),
        grid=(rows // blk,),
        in_specs=[pl.BlockSpec((blk, H), lambda i: (i, 0))],
        out_specs=pl.BlockSpec((blk, H // 2), lambda i: (i, 0)),
    )(tables.reshape(rows, H))


def kernel(input_ids, tables):
    B, L, S = input_ids.shape
    _, V, H = tables.shape
    R = B * S
    C = 8
    ids32 = input_ids.astype(jnp.int32)
    flat_idx = ids32 + (jnp.arange(L, dtype=jnp.int32) * V)[None, :, None]
    flat_idx = flat_idx.transpose(0, 2, 1).reshape(R // C, C * L)
    tab = _pack_tables(tables, L, V, H)
    out = _make_sc_kernel(R, H, L, C)(flat_idx, tab)
    return out.reshape(B, S, H)
